# Initial kernel scaffold; baseline (speedup 1.0000x reference)
import jax
import jax.numpy as jnp
from jax.experimental import pallas as pl


def kernel(edge_attr, W0, W1, W2):
    raise NotImplementedError("write your pallas kernel here")


# speedup vs baseline: 7.4907x; 7.4907x over previous
import jax
import jax.numpy as jnp
from jax.experimental import pallas as pl
from jax.experimental.pallas import tpu as pltpu

E = 320000
D = 128
BE = 2560


def _tc_body(idx_ref, w0_ref, w1_ref, w2_ref, out_ref):
    idx = idx_ref[...]
    acc = None
    for col, w_ref in enumerate((w0_ref, w1_ref, w2_ref)):
        w = w_ref[...]
        rows = w.shape[0]
        oh = (idx[:, col:col + 1] == jax.lax.broadcasted_iota(
            jnp.int32, (1, rows), 1)).astype(jnp.float32)
        part = jax.lax.dot_general(
            oh, w, (((1,), (0,)), ((), ())),
            preferred_element_type=jnp.float32)
        acc = part if acc is None else acc + part
    out_ref[...] = acc


def kernel(edge_attr, W0, W1, W2):
    grid = (E // BE,)
    return pl.pallas_call(
        _tc_body,
        grid=grid,
        in_specs=[
            pl.BlockSpec((BE, 3), lambda i: (i, 0)),
            pl.BlockSpec((W0.shape[0], D), lambda i: (0, 0)),
            pl.BlockSpec((W1.shape[0], D), lambda i: (0, 0)),
            pl.BlockSpec((W2.shape[0], D), lambda i: (0, 0)),
        ],
        out_specs=pl.BlockSpec((BE, D), lambda i: (i, 0)),
        out_shape=jax.ShapeDtypeStruct((E, D), jnp.float32),
    )(edge_attr, W0, W1, W2)
